# baseline (speedup 1.0000x reference)
import functools

import jax
import jax.numpy as jnp
from jax import lax
from jax.experimental import pallas as pl
from jax.experimental.pallas import tpu as pltpu
from jax.experimental.pallas import tpu_sc as plsc

B = 16
D = 768
MAXD = 512
NDOCS = B * MAXD
NW = 32
CHUNK = 64
OUT_PAD = 16
INV_T = 50.0


NGRP = CHUNK // 16


def _sc_body(
    q_hbm, docs_hbm, nd_hbm, out_hbm, qbuf, ndbuf, dbuf, sbuf, scorebuf, s0, s1, s2, s3
):
    sems = [s0, s1, s2, s3]
    wid = lax.axis_index("s") * 2 + lax.axis_index("c")

    pltpu.sync_copy(q_hbm, qbuf)
    pltpu.sync_copy(nd_hbm, ndbuf)

    nd_vec = ndbuf[...]
    csum_s, nd_s = [], []
    run = jnp.int32(0)
    for j in range(B):
        v = nd_vec[j]
        nd_s.append(v)
        run = run + v
        csum_s.append(run)
    total = run

    t0 = (wid * total) // NW
    t1 = ((wid + 1) * total) // NW
    ab0 = 8 * (t0 // 8)
    nchunks = jnp.where(t1 > t0, (t1 - ab0 + CHUNK - 1) // CHUNK, 0)

    iota = lax.iota(jnp.int32, 16)
    one = jnp.ones((16,), jnp.int32)
    zero = jnp.zeros((16,), jnp.int32)
    fzero = jnp.zeros((16,), jnp.float32)

    def chunk_body(c, _):
        abase = ab0 + c * CHUNK
        cb = jnp.minimum(abase, NDOCS - CHUNK)
        pltpu.sync_copy(docs_hbm.at[pl.ds(cb * D, CHUNK * D)], dbuf)
        handles = []
        for g in range(NGRP):
            t_vec = abase + g * 16 + iota
            mask = (t_vec >= one * t0) & (t_vec < one * t1)
            b_vec = zero
            off_vec = zero
            for j in range(B):
                cond = t_vec >= one * csum_s[j]
                b_vec = b_vec + jnp.where(cond, one, zero)
                off_vec = off_vec + jnp.where(cond, one * nd_s[j], zero)
            b_vec = jnp.minimum(b_vec, B - 1)
            outidx = jnp.where(mask, b_vec * MAXD + (t_vec - off_vec), NDOCS + iota)
            rows_vec = jnp.where(mask, t_vec - cb, zero)
            qoff_vec = b_vec * D
            roff_vec = rows_vec * D
            for j in range(16):
                qb = qoff_vec[j]
                rb = roff_vec[j]

                def kc_body(kc, acc, qb=qb, rb=rb):
                    base = kc * 256
                    for u in range(16):
                        qv = qbuf[pl.ds(qb + (base + u * 16), 16)]
                        dv = dbuf[pl.ds(rb + (base + u * 16), 16)]
                        acc = acc + qv * dv
                    return acc

                acc = lax.fori_loop(0, D // 256, kc_body, fzero)
                sbuf[pl.ds(j * 17, 16)] = acc
            svec = fzero
            for l in range(16):
                svec = svec + plsc.load_gather(sbuf, [iota * 17 + l])
            scorebuf[pl.ds(g * 16, 16)] = svec
            handles.append(
                pltpu.async_copy(
                    scorebuf.at[pl.ds(g * 16, 16)], out_hbm.at[outidx], sems[g]
                )
            )
        for h in handles:
            h.wait()
        return 0

    lax.fori_loop(0, nchunks, chunk_body, 0)


def _sc_sim(q_flat, doc_flat, nd):
    kfn = functools.partial(
        pl.kernel,
        mesh=plsc.VectorSubcoreMesh(core_axis_name="c", subcore_axis_name="s"),
        compiler_params=pltpu.CompilerParams(needs_layout_passes=False),
        out_type=jax.ShapeDtypeStruct((NDOCS + OUT_PAD,), jnp.float32),
        scratch_types=[
            pltpu.VMEM((B * D,), jnp.float32),
            pltpu.VMEM((B,), jnp.int32),
            pltpu.VMEM((CHUNK * D,), jnp.float32),
            pltpu.VMEM((16 * 17,), jnp.float32),
            pltpu.VMEM((CHUNK,), jnp.float32),
            pltpu.SemaphoreType.DMA,
            pltpu.SemaphoreType.DMA,
            pltpu.SemaphoreType.DMA,
            pltpu.SemaphoreType.DMA,
        ],
    )(_sc_body)
    return kfn(q_flat, doc_flat, nd)


def _tc_body(nd_smem, sim_ref, labels_ref, ndv_ref, out_ref):
    sim = sim_ref[...] * INV_T
    ndcol = ndv_ref[...]
    pos = lax.broadcasted_iota(jnp.int32, (B, MAXD), 1)
    mask = pos < ndcol
    sims = jnp.where(mask, sim, -jnp.inf)
    mx = jnp.max(sims, axis=1, keepdims=True)
    mxs = jnp.where(ndcol > 0, mx, 0.0)
    ex = jnp.where(mask, jnp.exp(sims - mxs), 0.0)
    sexp = jnp.sum(ex, axis=1, keepdims=True)
    logz = jnp.log(sexp)

    labels = labels_ref[...]
    pt = jnp.where(mask, labels, 0.0)
    s = jnp.sum(pt, axis=1, keepdims=True) + 1e-9
    pt = pt / s
    logpt = jnp.log(jnp.where(pt > 0, pt, 1.0))
    logsm = sims - mxs - logz
    terms = jnp.where(mask, pt * logpt - pt * logsm, 0.0)
    out_ref[0, 0] = jnp.sum(terms) * (1.0 / B)


def _tc_loss(sim2d, soft_labels, nd):
    return pl.pallas_call(
        _tc_body,
        in_specs=[
            pl.BlockSpec(memory_space=pltpu.SMEM),
            pl.BlockSpec((B, MAXD), lambda: (0, 0)),
            pl.BlockSpec((B, MAXD), lambda: (0, 0)),
            pl.BlockSpec((B, 1), lambda: (0, 0)),
        ],
        out_specs=pl.BlockSpec(memory_space=pltpu.SMEM),
        out_shape=jax.ShapeDtypeStruct((1, 1), jnp.float32),
    )(nd, sim2d, soft_labels, nd.reshape(B, 1))


def kernel(query_embeds, doc_embeds, soft_labels, num_docs_per_sample):
    nd = num_docs_per_sample.astype(jnp.int32)
    q_flat = query_embeds.reshape(-1)
    simflat = _sc_sim(q_flat, doc_embeds.reshape(-1), nd)
    sim2d = simflat[:NDOCS].reshape(B, MAXD)
    out = _tc_loss(sim2d, soft_labels, nd)
    return out[0, 0]


# speedup vs baseline: 4.5733x; 4.5733x over previous
import functools

import jax
import jax.numpy as jnp
from jax import lax
from jax.experimental import pallas as pl
from jax.experimental.pallas import tpu as pltpu
from jax.experimental.pallas import tpu_sc as plsc

B = 16
D = 768
MAXD = 512
NDOCS = B * MAXD
NCHUNK = NDOCS // MAXD + 1
SROWS = NCHUNK * MAXD
HALF = MAXD // 2
INV_T = 50.0


def _mm_body(nlive_ref, doc_ref, qt_ref, out_ref):
    out_ref[...] = jnp.dot(
        doc_ref[...], qt_ref[...], preferred_element_type=jnp.float32
    )


def _tc_scores(doc_embeds, qt, nlive_m1):
    return pl.pallas_call(
        _mm_body,
        grid_spec=pltpu.PrefetchScalarGridSpec(
            num_scalar_prefetch=1,
            grid=(NCHUNK,),
            in_specs=[
                pl.BlockSpec((MAXD, D), lambda i, n: (jnp.minimum(i, n[0]), 0)),
                pl.BlockSpec((D, B), lambda i, n: (0, 0)),
            ],
            out_specs=pl.BlockSpec((MAXD, B), lambda i, n: (i, 0)),
        ),
        out_shape=jax.ShapeDtypeStruct((SROWS, B), jnp.float32),
    )(nlive_m1, doc_embeds, qt)


def _sc_body(s_hbm, nd_hbm, out_hbm, ndbuf, dbuf, outbuf):
    b = lax.axis_index("s")
    half = lax.axis_index("c")

    pltpu.sync_copy(nd_hbm, ndbuf)
    nd_vec = ndbuf[...]
    iota = lax.iota(jnp.int32, 16)
    offs_vec = plsc.cumsum(nd_vec) - nd_vec
    off_b = jnp.sum(jnp.where(iota == b, offs_vec, 0))

    src = (off_b + half * HALF) * B
    pltpu.sync_copy(s_hbm.at[pl.ds(src, HALF * B)], dbuf)

    for g in range(HALF // 16):
        idx = g * 256 + iota * B + b
        outbuf[pl.ds(g * 16, 16)] = plsc.load_gather(dbuf, [idx])
    pltpu.sync_copy(outbuf, out_hbm.at[pl.ds(b * MAXD + half * HALF, HALF)])


def _sc_extract(s_flat, nd):
    kfn = functools.partial(
        pl.kernel,
        mesh=plsc.VectorSubcoreMesh(core_axis_name="c", subcore_axis_name="s"),
        compiler_params=pltpu.CompilerParams(needs_layout_passes=False),
        out_type=jax.ShapeDtypeStruct((NDOCS,), jnp.float32),
        scratch_types=[
            pltpu.VMEM((B,), jnp.int32),
            pltpu.VMEM((HALF * B,), jnp.float32),
            pltpu.VMEM((HALF,), jnp.float32),
        ],
    )(_sc_body)
    return kfn(s_flat, nd)


def _tc_body(nd_smem, sim_ref, labels_ref, ndv_ref, out_ref):
    sim = sim_ref[...] * INV_T
    ndcol = ndv_ref[...]
    pos = lax.broadcasted_iota(jnp.int32, (B, MAXD), 1)
    mask = pos < ndcol
    sims = jnp.where(mask, sim, -jnp.inf)
    mx = jnp.max(sims, axis=1, keepdims=True)
    mxs = jnp.where(ndcol > 0, mx, 0.0)
    ex = jnp.where(mask, jnp.exp(sims - mxs), 0.0)
    sexp = jnp.sum(ex, axis=1, keepdims=True)
    logz = jnp.log(sexp)

    labels = labels_ref[...]
    pt = jnp.where(mask, labels, 0.0)
    s = jnp.sum(pt, axis=1, keepdims=True) + 1e-9
    pt = pt / s
    logpt = jnp.log(jnp.where(pt > 0, pt, 1.0))
    logsm = sims - mxs - logz
    terms = jnp.where(mask, pt * logpt - pt * logsm, 0.0)
    out_ref[0, 0] = jnp.sum(terms) * (1.0 / B)


def _tc_loss(sim2d, soft_labels, nd):
    return pl.pallas_call(
        _tc_body,
        in_specs=[
            pl.BlockSpec(memory_space=pltpu.SMEM),
            pl.BlockSpec((B, MAXD), lambda: (0, 0)),
            pl.BlockSpec((B, MAXD), lambda: (0, 0)),
            pl.BlockSpec((B, 1), lambda: (0, 0)),
        ],
        out_specs=pl.BlockSpec(memory_space=pltpu.SMEM),
        out_shape=jax.ShapeDtypeStruct((1, 1), jnp.float32),
    )(nd, sim2d, soft_labels, nd.reshape(B, 1))


def kernel(query_embeds, doc_embeds, soft_labels, num_docs_per_sample):
    nd = num_docs_per_sample.astype(jnp.int32)
    total = jnp.sum(nd)
    nlive_m1 = jnp.maximum((total + MAXD - 1) // MAXD - 1, 0).reshape(1)
    s = _tc_scores(doc_embeds, query_embeds.T, nlive_m1)
    simflat = _sc_extract(s.reshape(-1), nd)
    sim2d = simflat.reshape(B, MAXD)
    out = _tc_loss(sim2d, soft_labels, nd)
    return out[0, 0]


# speedup vs baseline: 5.5024x; 1.2031x over previous
import functools

import jax
import jax.numpy as jnp
from jax import lax
from jax.experimental import pallas as pl
from jax.experimental.pallas import tpu as pltpu
from jax.experimental.pallas import tpu_sc as plsc

B = 16
D = 768
MAXD = 512
NDOCS = B * MAXD
NCHUNK = NDOCS // MAXD + 1
SROWS = NCHUNK * MAXD
HALF = MAXD // 2
INV_T = 50.0


def _mm_body(nd_ref, q_ref, doc_ref, out_ref):
    out_ref[...] = lax.dot_general(
        q_ref[...],
        doc_ref[...],
        (((1,), (1,)), ((), ())),
        preferred_element_type=jnp.float32,
    )


def _doc_map(i, nd):
    total = nd[0]
    for k in range(1, B):
        total = total + nd[k]
    nlive_m1 = jnp.maximum((total + MAXD - 1) // MAXD - 1, 0)
    return (jnp.minimum(i, nlive_m1), 0)


def _tc_scores(q, doc_embeds, nd):
    return pl.pallas_call(
        _mm_body,
        grid_spec=pltpu.PrefetchScalarGridSpec(
            num_scalar_prefetch=1,
            grid=(NCHUNK,),
            in_specs=[
                pl.BlockSpec((B, D), lambda i, nd: (0, 0)),
                pl.BlockSpec((MAXD, D), _doc_map),
            ],
            out_specs=pl.BlockSpec((B, MAXD), lambda i, nd: (0, i)),
        ),
        out_shape=jax.ShapeDtypeStruct((B, SROWS), jnp.float32),
    )(nd, q, doc_embeds)


def _sc_body(st_hbm, nd_hbm, out_hbm, ndbuf, dbuf, outbuf):
    b = lax.axis_index("s")
    half = lax.axis_index("c")

    pltpu.sync_copy(nd_hbm, ndbuf)
    nd_vec = ndbuf[...]
    iota = lax.iota(jnp.int32, 16)
    offs_vec = plsc.cumsum(nd_vec) - nd_vec
    off_b = jnp.sum(jnp.where(iota == b, offs_vec, 0))

    start = b * SROWS + off_b + half * HALF
    astart = (start // 16) * 16
    m = start - astart
    pltpu.sync_copy(st_hbm.at[pl.ds(astart, HALF + 16)], dbuf)

    for g in range(HALF // 16):
        outbuf[pl.ds(g * 16, 16)] = plsc.load_gather(dbuf, [m + g * 16 + iota])
    pltpu.sync_copy(outbuf, out_hbm.at[pl.ds(b * MAXD + half * HALF, HALF)])


def _sc_extract(st_flat, nd):
    kfn = functools.partial(
        pl.kernel,
        mesh=plsc.VectorSubcoreMesh(core_axis_name="c", subcore_axis_name="s"),
        compiler_params=pltpu.CompilerParams(needs_layout_passes=False),
        out_type=jax.ShapeDtypeStruct((NDOCS,), jnp.float32),
        scratch_types=[
            pltpu.VMEM((B,), jnp.int32),
            pltpu.VMEM((HALF + 16,), jnp.float32),
            pltpu.VMEM((HALF,), jnp.float32),
        ],
    )(_sc_body)
    return kfn(st_flat, nd)


def _tc_body(nd_smem, sim_ref, labels_ref, ndv_ref, out_ref):
    sim = sim_ref[...] * INV_T
    ndcol = ndv_ref[...]
    pos = lax.broadcasted_iota(jnp.int32, (B, MAXD), 1)
    mask = pos < ndcol
    sims = jnp.where(mask, sim, -jnp.inf)
    mx = jnp.max(sims, axis=1, keepdims=True)
    mxs = jnp.where(ndcol > 0, mx, 0.0)
    ex = jnp.where(mask, jnp.exp(sims - mxs), 0.0)
    sexp = jnp.sum(ex, axis=1, keepdims=True)
    logz = jnp.log(sexp)

    labels = labels_ref[...]
    pt = jnp.where(mask, labels, 0.0)
    s = jnp.sum(pt, axis=1, keepdims=True) + 1e-9
    pt = pt / s
    logpt = jnp.log(jnp.where(pt > 0, pt, 1.0))
    logsm = sims - mxs - logz
    terms = jnp.where(mask, pt * logpt - pt * logsm, 0.0)
    out_ref[0, 0] = jnp.sum(terms) * (1.0 / B)


def _tc_loss(sim2d, soft_labels, nd):
    return pl.pallas_call(
        _tc_body,
        in_specs=[
            pl.BlockSpec(memory_space=pltpu.SMEM),
            pl.BlockSpec((B, MAXD), lambda: (0, 0)),
            pl.BlockSpec((B, MAXD), lambda: (0, 0)),
            pl.BlockSpec((B, 1), lambda: (0, 0)),
        ],
        out_specs=pl.BlockSpec(memory_space=pltpu.SMEM),
        out_shape=jax.ShapeDtypeStruct((1, 1), jnp.float32),
    )(nd, sim2d, soft_labels, nd.reshape(B, 1))


def kernel(query_embeds, doc_embeds, soft_labels, num_docs_per_sample):
    nd = num_docs_per_sample.astype(jnp.int32)
    st = _tc_scores(query_embeds, doc_embeds, nd)
    simflat = _sc_extract(st.reshape(-1), nd)
    sim2d = simflat.reshape(B, MAXD)
    out = _tc_loss(sim2d, soft_labels, nd)
    return out[0, 0]


# speedup vs baseline: 5.7262x; 1.0407x over previous
import functools

import jax
import jax.numpy as jnp
from jax import lax
from jax.experimental import pallas as pl
from jax.experimental.pallas import tpu as pltpu
from jax.experimental.pallas import tpu_sc as plsc

B = 16
D = 768
MAXD = 512
NDOCS = B * MAXD
NCHUNK = NDOCS // MAXD + 1
SROWS = NCHUNK * MAXD
HALF = MAXD // 2
INV_T = 50.0


def _mm_body(nd_ref, q_ref, doc_ref, out_ref):
    out_ref[...] = lax.dot_general(
        q_ref[...],
        doc_ref[...],
        (((1,), (1,)), ((), ())),
        preferred_element_type=jnp.float32,
    )


def _doc_map(i, nd):
    total = nd[0]
    for k in range(1, B):
        total = total + nd[k]
    nlive_m1 = jnp.maximum((total + MAXD - 1) // MAXD - 1, 0)
    return (jnp.minimum(i, nlive_m1), 0)


def _out_map(i, nd):
    total = nd[0]
    for k in range(1, B):
        total = total + nd[k]
    nlive_m1 = jnp.maximum((total + MAXD - 1) // MAXD - 1, 0)
    return (0, jnp.minimum(i, nlive_m1))


def _tc_scores(q, doc_embeds, nd):
    return pl.pallas_call(
        _mm_body,
        grid_spec=pltpu.PrefetchScalarGridSpec(
            num_scalar_prefetch=1,
            grid=(NCHUNK,),
            in_specs=[
                pl.BlockSpec((B, D), lambda i, nd: (0, 0)),
                pl.BlockSpec((MAXD, D), _doc_map),
            ],
            out_specs=pl.BlockSpec((B, MAXD), _out_map),
        ),
        out_shape=jax.ShapeDtypeStruct((B, SROWS), jnp.float32),
    )(nd, q, doc_embeds)


def _sc_body(st_hbm, nd_hbm, out_hbm, ndbuf, dbuf, outbuf):
    b = lax.axis_index("s")
    half = lax.axis_index("c")

    pltpu.sync_copy(nd_hbm, ndbuf)
    nd_vec = ndbuf[...]
    iota = lax.iota(jnp.int32, 16)
    offs_vec = plsc.cumsum(nd_vec) - nd_vec
    off_b = jnp.sum(jnp.where(iota == b, offs_vec, 0))

    start = b * SROWS + off_b + half * HALF
    astart = (start // 16) * 16
    m = start - astart
    pltpu.sync_copy(st_hbm.at[pl.ds(astart, HALF + 16)], dbuf)

    for g in range(HALF // 16):
        outbuf[pl.ds(g * 16, 16)] = plsc.load_gather(dbuf, [m + g * 16 + iota])
    pltpu.sync_copy(outbuf, out_hbm.at[pl.ds(b * MAXD + half * HALF, HALF)])


def _sc_extract(st_flat, nd):
    kfn = functools.partial(
        pl.kernel,
        mesh=plsc.VectorSubcoreMesh(core_axis_name="c", subcore_axis_name="s"),
        compiler_params=pltpu.CompilerParams(needs_layout_passes=False),
        out_type=jax.ShapeDtypeStruct((NDOCS,), jnp.float32),
        scratch_types=[
            pltpu.VMEM((B,), jnp.int32),
            pltpu.VMEM((HALF + 16,), jnp.float32),
            pltpu.VMEM((HALF,), jnp.float32),
        ],
    )(_sc_body)
    return kfn(st_flat, nd)


def _tc_body(nd_smem, sim_ref, labels_ref, ndv_ref, out_ref):
    sim = sim_ref[...] * INV_T
    ndcol = ndv_ref[...]
    pos = lax.broadcasted_iota(jnp.int32, (B, MAXD), 1)
    mask = pos < ndcol
    sims = jnp.where(mask, sim, -jnp.inf)
    mx = jnp.max(sims, axis=1, keepdims=True)
    mxs = jnp.where(ndcol > 0, mx, 0.0)
    ex = jnp.where(mask, jnp.exp(sims - mxs), 0.0)
    sexp = jnp.sum(ex, axis=1, keepdims=True)
    logz = jnp.log(sexp)

    labels = labels_ref[...]
    pt = jnp.where(mask, labels, 0.0)
    s = jnp.sum(pt, axis=1, keepdims=True) + 1e-9
    pt = pt / s
    logpt = jnp.log(jnp.where(pt > 0, pt, 1.0))
    logsm = sims - mxs - logz
    terms = jnp.where(mask, pt * logpt - pt * logsm, 0.0)
    out_ref[0, 0] = jnp.sum(terms) * (1.0 / B)


def _tc_loss(sim2d, soft_labels, nd):
    return pl.pallas_call(
        _tc_body,
        in_specs=[
            pl.BlockSpec(memory_space=pltpu.SMEM),
            pl.BlockSpec((B, MAXD), lambda: (0, 0)),
            pl.BlockSpec((B, MAXD), lambda: (0, 0)),
            pl.BlockSpec((B, 1), lambda: (0, 0)),
        ],
        out_specs=pl.BlockSpec(memory_space=pltpu.SMEM),
        out_shape=jax.ShapeDtypeStruct((1, 1), jnp.float32),
    )(nd, sim2d, soft_labels, nd.reshape(B, 1))


def kernel(query_embeds, doc_embeds, soft_labels, num_docs_per_sample):
    nd = num_docs_per_sample.astype(jnp.int32)
    st = _tc_scores(query_embeds, doc_embeds, nd)
    simflat = _sc_extract(st.reshape(-1), nd)
    sim2d = simflat.reshape(B, MAXD)
    out = _tc_loss(sim2d, soft_labels, nd)
    return out[0, 0]
